# baseline (speedup 1.0000x reference)
import functools

import numpy as np
import jax
import jax.numpy as jnp
from jax import lax
from jax.experimental import pallas as pl
from jax.experimental.pallas import tpu as pltpu
from jax.experimental.pallas import tpu_sc as plsc

_DIM = 3
_L = 16
_F = 2
_BASE = 16
_SCALE = 1.38191
_T = 1 << 19
_N = 262144
_MASK = _T - 1
_P1 = -1640531535
_P2 = 805459861

_RES = tuple(int(np.floor(_BASE * _SCALE**l)) for l in range(_L))

_NC = 2
_NS = 16
_NW = _NC * _NS
_PW = _N // _NW
_C = 2048
_NCHUNK = _PW // _C


def _encoder_body(xt, tbl0, tbl1, resf, out,
                  xv, frac, idxb0, idxb1, fa0, fa1, fb0, fb1, outb,
                  sem0, sem1, osem0, osem1):
    del resf
    wid = lax.axis_index("s") * _NC + lax.axis_index("c")
    sems = (sem0, sem1)
    osems = (osem0, osem1)
    idxbs = (idxb0, idxb1)
    fas = (fa0, fa1)
    fbs = (fb0, fb1)

    def chunk_body(ci, _):
        base = wid * _PW + ci * _C
        pltpu.sync_copy(xt.at[:, pl.ds(base, _C)], xv)

        def p1(l):
            res = float(_RES[l])
            lT = l * _T
            b = l % 2

            def body(v, _):
                s = v * 16
                x0 = xv[0, pl.ds(s, 16)]
                x1 = xv[1, pl.ds(s, 16)]
                x2 = xv[2, pl.ds(s, 16)]

                def split(xdd):
                    pos = xdd * res
                    p0 = pos.astype(jnp.int32)
                    return p0, pos - p0.astype(jnp.float32)

                pa, fa = split(x0)
                pb, fb = split(x1)
                pc, fc = split(x2)
                frac[b, 0, pl.ds(s, 16)] = fa
                frac[b, 1, pl.ds(s, 16)] = fb
                frac[b, 2, pl.ds(s, 16)] = fc
                h0 = (pa, pa + 1)
                h1 = (pb * _P1, pb * _P1 + _P1)
                h2 = (pc * _P2, pc * _P2 + _P2)
                for c in range(8):
                    h = h0[c >> 2] ^ h1[(c >> 1) & 1] ^ h2[c & 1]
                    idxbs[b][pl.ds(c * _C + s, 16)] = (h & _MASK) + lT
                return 0

            lax.fori_loop(0, _C // 16, body, 0, unroll=False)

        def fire(l):
            b = l % 2
            return (
                pltpu.async_copy(tbl0.at[idxbs[b]], fas[b], sems[b]),
                pltpu.async_copy(tbl1.at[idxbs[b]], fbs[b], sems[b]),
            )

        def p2(l):
            b = l % 2

            def body(v, _):
                s = v * 16
                fx = frac[b, 0, pl.ds(s, 16)]
                fy = frac[b, 1, pl.ds(s, 16)]
                fz = frac[b, 2, pl.ds(s, 16)]
                g = [
                    (fas[b][pl.ds(c * _C + s, 16)],
                     fbs[b][pl.ds(c * _C + s, 16)])
                    for c in range(8)
                ]

                def lerp(a, bb, t):
                    return a + t * (bb - a)

                for j in range(_F):
                    v00 = lerp(g[0][j], g[1][j], fz)
                    v01 = lerp(g[2][j], g[3][j], fz)
                    v10 = lerp(g[4][j], g[5][j], fz)
                    v11 = lerp(g[6][j], g[7][j], fz)
                    outb[b, j, pl.ds(s, 16)] = lerp(
                        lerp(v00, v01, fy), lerp(v10, v11, fy), fx
                    )
                return 0

            lax.fori_loop(0, _C // 16, body, 0, unroll=False)

        copies = {}
        outcp = {}
        p1(0)
        copies[0] = fire(0)
        for l in range(_L):
            if l + 1 < _L:
                p1(l + 1)
                copies[l + 1] = fire(l + 1)
            cpa, cpb = copies.pop(l)
            cpa.wait()
            cpb.wait()
            if l >= 2:
                outcp.pop(l - 2).wait()
            p2(l)
            outcp[l] = pltpu.async_copy(
                outb.at[l % 2], out.at[l, :, pl.ds(base, _C)], osems[l % 2]
            )
        outcp.pop(_L - 2).wait()
        outcp.pop(_L - 1).wait()
        return 0

    lax.fori_loop(0, _NCHUNK, chunk_body, 0, unroll=False)


@jax.jit
def kernel(x, table):
    xt = x.T
    tbl0 = table[:, :, 0].reshape(_L * _T)
    tbl1 = table[:, :, 1].reshape(_L * _T)
    resf = jnp.zeros((8,), jnp.float32)

    mesh = plsc.VectorSubcoreMesh(core_axis_name="c", subcore_axis_name="s")
    enc = functools.partial(
        pl.kernel,
        out_type=jax.ShapeDtypeStruct((_L, _F, _N), jnp.float32),
        mesh=mesh,
        scratch_types=[
            pltpu.VMEM((_DIM, _C), jnp.float32),
            pltpu.VMEM((2, _DIM, _C), jnp.float32),
            pltpu.VMEM((8 * _C,), jnp.int32),
            pltpu.VMEM((8 * _C,), jnp.int32),
            pltpu.VMEM((8 * _C,), jnp.float32),
            pltpu.VMEM((8 * _C,), jnp.float32),
            pltpu.VMEM((8 * _C,), jnp.float32),
            pltpu.VMEM((8 * _C,), jnp.float32),
            pltpu.VMEM((2, _F, _C), jnp.float32),
            pltpu.SemaphoreType.DMA,
            pltpu.SemaphoreType.DMA,
            pltpu.SemaphoreType.DMA,
            pltpu.SemaphoreType.DMA,
        ],
    )(_encoder_body)
    out = enc(xt, tbl0, tbl1, resf)
    return out.transpose(2, 0, 1).reshape(_N, _L * _F)


# speedup vs baseline: 1.0146x; 1.0146x over previous
import functools

import numpy as np
import jax
import jax.numpy as jnp
from jax import lax
from jax.experimental import pallas as pl
from jax.experimental.pallas import tpu as pltpu
from jax.experimental.pallas import tpu_sc as plsc

_DIM = 3
_L = 16
_F = 2
_BASE = 16
_SCALE = 1.38191
_T = 1 << 19
_N = 262144
_MASK = _T - 1
_P1 = -1640531535
_P2 = 805459861

_RES = tuple(int(np.floor(_BASE * _SCALE**l)) for l in range(_L))

_NC = 2
_NS = 16
_NW = _NC * _NS
_PW = _N // _NW
_C = 1024
_NCHUNK = _PW // _C


def _encoder_body(xt, tbl0, tbl1, resf, out,
                  xv, frac, idxb0, idxb1, fa0, fa1, fb0, fb1, outb,
                  sem0, sem1, osem0, osem1):
    del resf
    wid = lax.axis_index("s") * _NC + lax.axis_index("c")
    sems = (sem0, sem1)
    osems = (osem0, osem1)
    idxbs = (idxb0, idxb1)
    fas = (fa0, fa1)
    fbs = (fb0, fb1)

    def chunk_body(ci, _):
        base = wid * _PW + ci * _C
        pltpu.sync_copy(xt.at[:, pl.ds(base, _C)], xv)

        def p1(l):
            res = float(_RES[l])
            lT = l * _T
            b = l % 2

            def body(v, _):
                s = v * 16
                x0 = xv[0, pl.ds(s, 16)]
                x1 = xv[1, pl.ds(s, 16)]
                x2 = xv[2, pl.ds(s, 16)]

                def split(xdd):
                    pos = xdd * res
                    p0 = pos.astype(jnp.int32)
                    return p0, pos - p0.astype(jnp.float32)

                pa, fa = split(x0)
                pb, fb = split(x1)
                pc, fc = split(x2)
                frac[b, 0, pl.ds(s, 16)] = fa
                frac[b, 1, pl.ds(s, 16)] = fb
                frac[b, 2, pl.ds(s, 16)] = fc
                h0 = (pa, pa + 1)
                h1 = (pb * _P1, pb * _P1 + _P1)
                h2 = (pc * _P2, pc * _P2 + _P2)
                for c in range(8):
                    h = h0[c >> 2] ^ h1[(c >> 1) & 1] ^ h2[c & 1]
                    idxbs[b][pl.ds(c * _C + s, 16)] = (h & _MASK) + lT
                return 0

            lax.fori_loop(0, _C // 16, body, 0, unroll=False)

        def fire(l):
            b = l % 2
            return (
                pltpu.async_copy(tbl0.at[idxbs[b]], fas[b], sems[b]),
                pltpu.async_copy(tbl1.at[idxbs[b]], fbs[b], sems[b]),
            )

        def p2(l):
            b = l % 2

            def body(v, _):
                s = v * 16
                fx = frac[b, 0, pl.ds(s, 16)]
                fy = frac[b, 1, pl.ds(s, 16)]
                fz = frac[b, 2, pl.ds(s, 16)]
                g = [
                    (fas[b][pl.ds(c * _C + s, 16)],
                     fbs[b][pl.ds(c * _C + s, 16)])
                    for c in range(8)
                ]

                def lerp(a, bb, t):
                    return a + t * (bb - a)

                for j in range(_F):
                    v00 = lerp(g[0][j], g[1][j], fz)
                    v01 = lerp(g[2][j], g[3][j], fz)
                    v10 = lerp(g[4][j], g[5][j], fz)
                    v11 = lerp(g[6][j], g[7][j], fz)
                    outb[b, j, pl.ds(s, 16)] = lerp(
                        lerp(v00, v01, fy), lerp(v10, v11, fy), fx
                    )
                return 0

            lax.fori_loop(0, _C // 16, body, 0, unroll=False)

        copies = {}
        outcp = {}
        p1(0)
        copies[0] = fire(0)
        for l in range(_L):
            if l + 1 < _L:
                p1(l + 1)
                copies[l + 1] = fire(l + 1)
            cpa, cpb = copies.pop(l)
            cpa.wait()
            cpb.wait()
            if l >= 2:
                outcp.pop(l - 2).wait()
            p2(l)
            outcp[l] = pltpu.async_copy(
                outb.at[l % 2], out.at[l, :, pl.ds(base, _C)], osems[l % 2]
            )
        outcp.pop(_L - 2).wait()
        outcp.pop(_L - 1).wait()
        return 0

    lax.fori_loop(0, _NCHUNK, chunk_body, 0, unroll=False)


@jax.jit
def kernel(x, table):
    xt = x.T
    tbl0 = table[:, :, 0].reshape(_L * _T)
    tbl1 = table[:, :, 1].reshape(_L * _T)
    resf = jnp.zeros((8,), jnp.float32)

    mesh = plsc.VectorSubcoreMesh(core_axis_name="c", subcore_axis_name="s")
    enc = functools.partial(
        pl.kernel,
        out_type=jax.ShapeDtypeStruct((_L, _F, _N), jnp.float32),
        mesh=mesh,
        scratch_types=[
            pltpu.VMEM((_DIM, _C), jnp.float32),
            pltpu.VMEM((2, _DIM, _C), jnp.float32),
            pltpu.VMEM((8 * _C,), jnp.int32),
            pltpu.VMEM((8 * _C,), jnp.int32),
            pltpu.VMEM((8 * _C,), jnp.float32),
            pltpu.VMEM((8 * _C,), jnp.float32),
            pltpu.VMEM((8 * _C,), jnp.float32),
            pltpu.VMEM((8 * _C,), jnp.float32),
            pltpu.VMEM((2, _F, _C), jnp.float32),
            pltpu.SemaphoreType.DMA,
            pltpu.SemaphoreType.DMA,
            pltpu.SemaphoreType.DMA,
            pltpu.SemaphoreType.DMA,
        ],
    )(_encoder_body)
    out = enc(xt, tbl0, tbl1, resf)
    return out.transpose(2, 0, 1).reshape(_N, _L * _F)
